# Initial kernel scaffold; baseline (speedup 1.0000x reference)
import jax
import jax.numpy as jnp
from jax.experimental import pallas as pl


def kernel(x, locs, logscales, logcoefs):
    raise NotImplementedError("write your pallas kernel here")


# speedup vs baseline: 1.6632x; 1.6632x over previous
import functools
import math

import jax
import jax.numpy as jnp
from jax.experimental import pallas as pl
from jax.experimental.pallas import tpu as pltpu

_B = 262144
_D = 64
_K = 8
_MIN_SCALE = 1e-10
_LOG2PI = math.log(2.0 * math.pi)

_BLK = 4096


def _tc_body(x_ref, locs_ref, logscales_ref, logcoefs_ref, out_ref):
    locs = locs_ref[...]
    logscales = logscales_ref[...]
    logcoefs = logcoefs_ref[...]

    scale = jnp.exp(logscales) + _MIN_SCALE
    inv2 = 1.0 / (scale * scale)
    a = -0.5 * inv2
    t = locs * inv2
    lc = logcoefs[0] - jax.nn.logsumexp(logcoefs[0])
    c = (lc
         - jnp.sum(jnp.log(scale), axis=1)
         - 0.5 * _D * _LOG2PI
         - 0.5 * jnp.sum(locs * locs * inv2, axis=1))

    x = x_ref[...]
    lp = (jnp.dot(x * x, a.T, preferred_element_type=jnp.float32)
          + jnp.dot(x, t.T, preferred_element_type=jnp.float32)
          + c[None, :])
    m = jnp.max(lp, axis=1)
    s = jnp.sum(jnp.exp(lp - m[:, None]), axis=1)
    out_ref[...] = m + jnp.log(s)


@jax.jit
def kernel(x, locs, logscales, logcoefs):
    grid = _B // _BLK
    return pl.pallas_call(
        _tc_body,
        grid=(grid,),
        in_specs=[
            pl.BlockSpec((_BLK, _D), lambda i: (i, 0)),
            pl.BlockSpec((_K, _D), lambda i: (0, 0)),
            pl.BlockSpec((_K, _D), lambda i: (0, 0)),
            pl.BlockSpec((1, _K), lambda i: (0, 0)),
        ],
        out_specs=pl.BlockSpec((_BLK,), lambda i: (i,)),
        out_shape=jax.ShapeDtypeStruct((_B,), jnp.float32),
    )(x, locs, logscales, logcoefs)


# speedup vs baseline: 2.7594x; 1.6591x over previous
import math

import jax
import jax.numpy as jnp
from jax import lax
from jax.experimental import pallas as pl

_B = 262144
_D = 64
_K = 8
_MIN_SCALE = 1e-10
_LOG2PI = math.log(2.0 * math.pi)

_BLK = 4096


def _tc_body(x_ref, locs_ref, logscales_ref, logcoefs_ref, out_ref):
    locs = locs_ref[...]
    logscales = logscales_ref[...]
    logcoefs = logcoefs_ref[...]

    scale = jnp.exp(logscales) + _MIN_SCALE
    inv2 = 1.0 / (scale * scale)
    a = -0.5 * inv2
    t = locs * inv2
    lc = logcoefs[0] - jax.nn.logsumexp(logcoefs[0])
    c = (lc
         - jnp.sum(jnp.log(scale), axis=1)
         - 0.5 * _D * _LOG2PI
         - 0.5 * jnp.sum(locs * locs * inv2, axis=1))

    x = x_ref[...]
    nt = (((1,), (1,)), ((), ()))
    lp = (lax.dot_general(t, x, nt, preferred_element_type=jnp.float32)
          + lax.dot_general(a, x * x, nt, preferred_element_type=jnp.float32)
          + c[:, None])
    m = jnp.max(lp, axis=0)
    s = jnp.sum(jnp.exp(lp - m[None, :]), axis=0)
    out_ref[...] = (m + jnp.log(s))[None, None, :]


@jax.jit
def kernel(x, locs, logscales, logcoefs):
    grid = _B // _BLK
    out2d = pl.pallas_call(
        _tc_body,
        grid=(grid,),
        in_specs=[
            pl.BlockSpec((_BLK, _D), lambda i: (i, 0)),
            pl.BlockSpec((_K, _D), lambda i: (0, 0)),
            pl.BlockSpec((_K, _D), lambda i: (0, 0)),
            pl.BlockSpec((1, _K), lambda i: (0, 0)),
        ],
        out_specs=pl.BlockSpec((1, 1, _BLK), lambda i: (i, 0, 0)),
        out_shape=jax.ShapeDtypeStruct((grid, 1, _BLK), jnp.float32),
    )(x, locs, logscales, logcoefs)
    return out2d.reshape(_B)


# speedup vs baseline: 3.3423x; 1.2113x over previous
import math

import jax
import jax.numpy as jnp
from jax import lax
from jax.experimental import pallas as pl

_B = 262144
_D = 64
_K = 8
_MIN_SCALE = 1e-10
_LOG2PI = math.log(2.0 * math.pi)

_BLK = 16384


def _tc_body(x_ref, locs_ref, logscales_ref, logcoefs_ref, out_ref):
    locs = locs_ref[...]
    logscales = logscales_ref[...]
    logcoefs = logcoefs_ref[...]

    scale = jnp.exp(logscales) + _MIN_SCALE
    inv2 = 1.0 / (scale * scale)
    a = -0.5 * inv2
    t = locs * inv2
    lc = logcoefs[0] - jax.nn.logsumexp(logcoefs[0])
    c = (lc
         - jnp.sum(jnp.log(scale), axis=1)
         - 0.5 * _D * _LOG2PI
         - 0.5 * jnp.sum(locs * locs * inv2, axis=1))

    x = x_ref[...]
    nt = (((1,), (1,)), ((), ()))
    lp = (lax.dot_general(t, x, nt, preferred_element_type=jnp.float32)
          + lax.dot_general(a, x * x, nt, preferred_element_type=jnp.float32)
          + c[:, None])
    m = jnp.max(lp, axis=0)
    s = jnp.sum(jnp.exp(lp - m[None, :]), axis=0)
    out_ref[...] = (m + jnp.log(s))[None, None, :]


@jax.jit
def kernel(x, locs, logscales, logcoefs):
    grid = _B // _BLK
    out2d = pl.pallas_call(
        _tc_body,
        grid=(grid,),
        in_specs=[
            pl.BlockSpec((_BLK, _D), lambda i: (i, 0)),
            pl.BlockSpec((_K, _D), lambda i: (0, 0)),
            pl.BlockSpec((_K, _D), lambda i: (0, 0)),
            pl.BlockSpec((1, _K), lambda i: (0, 0)),
        ],
        out_specs=pl.BlockSpec((1, 1, _BLK), lambda i: (i, 0, 0)),
        out_shape=jax.ShapeDtypeStruct((grid, 1, _BLK), jnp.float32),
    )(x, locs, logscales, logcoefs)
    return out2d.reshape(_B)


# speedup vs baseline: 3.4061x; 1.0191x over previous
import math

import jax
import jax.numpy as jnp
from jax import lax
from jax.experimental import pallas as pl

_B = 262144
_D = 64
_K = 8
_MIN_SCALE = 1e-10
_LOG2PI = math.log(2.0 * math.pi)

_BLK = 32768


def _tc_body(x_ref, locs_ref, logscales_ref, logcoefs_ref, out_ref):
    locs = locs_ref[...]
    logscales = logscales_ref[...]
    logcoefs = logcoefs_ref[...]

    scale = jnp.exp(logscales) + _MIN_SCALE
    inv2 = 1.0 / (scale * scale)
    a = -0.5 * inv2
    t = locs * inv2
    lc = logcoefs[0] - jax.nn.logsumexp(logcoefs[0])
    c = (lc
         - jnp.sum(jnp.log(scale), axis=1)
         - 0.5 * _D * _LOG2PI
         - 0.5 * jnp.sum(locs * locs * inv2, axis=1))

    x = x_ref[...]
    nt = (((1,), (1,)), ((), ()))
    lp = (lax.dot_general(t, x, nt, preferred_element_type=jnp.float32)
          + lax.dot_general(a, x * x, nt, preferred_element_type=jnp.float32)
          + c[:, None])
    m = jnp.max(lp, axis=0)
    s = jnp.sum(jnp.exp(lp - m[None, :]), axis=0)
    out_ref[...] = (m + jnp.log(s))[None, None, :]


@jax.jit
def kernel(x, locs, logscales, logcoefs):
    grid = _B // _BLK
    out2d = pl.pallas_call(
        _tc_body,
        grid=(grid,),
        in_specs=[
            pl.BlockSpec((_BLK, _D), lambda i: (i, 0)),
            pl.BlockSpec((_K, _D), lambda i: (0, 0)),
            pl.BlockSpec((_K, _D), lambda i: (0, 0)),
            pl.BlockSpec((1, _K), lambda i: (0, 0)),
        ],
        out_specs=pl.BlockSpec((1, 1, _BLK), lambda i: (i, 0, 0)),
        out_shape=jax.ShapeDtypeStruct((grid, 1, _BLK), jnp.float32),
    )(x, locs, logscales, logcoefs)
    return out2d.reshape(_B)
